# baseline (speedup 1.0000x reference)
import functools

import jax
import jax.numpy as jnp
from jax import lax
from jax.experimental import pallas as pl
from jax.experimental.pallas import tpu as pltpu
from jax.experimental.pallas import tpu_sc as plsc

B = 32
T = 8192
D = 1024
KSEL = 256
L = 16
NC, NS = 2, 16
NW = NC * NS
NV = T // L
KV = KSEL // L
CH = 64
NCH = KSEL // CH
UNROLL = 8
HI_BITS = 7
INT_MIN = -2147483648

_mesh = plsc.VectorSubcoreMesh(
    core_axis_name="c", subcore_axis_name="s", num_cores=NC, num_subcores=NS
)


def _ones_where(m):
    return jnp.where(m, jnp.int32(1), jnp.int32(0))


@functools.partial(
    pl.kernel,
    out_type=(
        jax.ShapeDtypeStruct((B, KSEL), jnp.float32),
        jax.ShapeDtypeStruct((B, KSEL, D), jnp.float32),
    ),
    mesh=_mesh,
    compiler_params=pltpu.CompilerParams(needs_layout_passes=False),
    scratch_types=[
        pltpu.VMEM((T,), jnp.float32),
        pltpu.VMEM((T,), jnp.int32),
        pltpu.VMEM((T + L,), jnp.int32),
        pltpu.VMEM((KSEL,), jnp.int32),
        pltpu.VMEM((KSEL,), jnp.int32),
        pltpu.VMEM((KSEL,), jnp.float32),
        pltpu.VMEM((KSEL,), jnp.int32),
        pltpu.VMEM((CH, D), jnp.float32),
        pltpu.SemaphoreType.DMA,
    ],
)
def _sc_kmin_gather(s_hbm, x_hbm, vals_out, rows_out,
                    srow, keys, cand, ckey, cidx, sval, srid,
                    rbuf, gsem):
    wid = lax.axis_index("s") * NC + lax.axis_index("c")

    pltpu.sync_copy(s_hbm.at[wid], srow)

    def build_keys(i, acc):
        f = srow[pl.ds(i * L, L)]
        f = jnp.where(f == 0.0, jnp.float32(0.0), f)
        bi = plsc.bitcast(f, jnp.int32)
        u = jnp.where(bi < 0, bi ^ jnp.int32(0x7FFFFFFF), bi)
        keys[pl.ds(i * L, L)] = u
        return acc + _ones_where(u < 0)

    with jax.named_scope("kmin_keys"):
        c_neg = jnp.sum(plsc.parallel_loop(
            0, NV, 1, unroll=UNROLL,
            carry=jnp.zeros((L,), jnp.int32))(build_keys))

    def count_where(cmp_fn):
        def body(i, acc):
            u = keys[pl.ds(i * L, L)]
            return acc + _ones_where(cmp_fn(u))

        acc = plsc.parallel_loop(0, NV, 1, unroll=UNROLL,
                                 carry=jnp.zeros((L,), jnp.int32))(body)
        return jnp.sum(acc)

    kth = jnp.int32(KSEL)
    p = jnp.where(kth <= c_neg, jnp.int32(INT_MIN), jnp.int32(0))
    kth = jnp.where(kth <= c_neg, kth, kth - c_neg)
    with jax.named_scope("kmin_descent_hi"):
        for t in range(30, 30 - HI_BITS, -1):
            pt = lax.shift_right_arithmetic(p, t)
            c0 = count_where(
                lambda u, pt=pt, t=t: lax.shift_right_arithmetic(u, t) == pt)
            keep = kth <= c0
            p = jnp.where(keep, p, p | jnp.int32(1 << t))
            kth = jnp.where(keep, kth, kth - c0)

    tcut = 30 - HI_BITS + 1
    pcut = lax.shift_right_arithmetic(p, tcut)

    def ccand(i, base):
        u = keys[pl.ds(i * L, L)]
        m = lax.shift_right_arithmetic(u, tcut) == pcut
        mi = _ones_where(m)
        pos = jnp.maximum(base + plsc.cumsum(mi) - 1, 0)
        plsc.store_scatter(cand, [pos], u, mask=m)
        return base + plsc.all_reduce_population_count(m)

    with jax.named_scope("kmin_compact_cand"):
        cnt_v = plsc.parallel_loop(
            0, NV, 1, unroll=UNROLL,
            carry=jnp.zeros((L,), jnp.int32))(ccand)
        plsc.store_scatter(cand, [cnt_v + lax.iota(jnp.int32, L)],
                           jnp.zeros((L,), jnp.int32)
                           + (p ^ jnp.int32(INT_MIN)))
    cnt = jnp.max(cnt_v)
    nv2 = lax.div(cnt + jnp.int32(L - 1), jnp.int32(L))
    with jax.named_scope("kmin_descent_lo"):
        for shift in range(tcut - 3, -1, -3):
            base3 = lax.shift_right_arithmetic(p, shift)

            def body(i, acc, base3=base3, shift=shift):
                u = cand[pl.ds(i * L, L)]
                us = lax.shift_right_arithmetic(u, shift)
                return tuple(
                    acc[j] + _ones_where(us == (base3 | j))
                    for j in range(7))

            accs = plsc.parallel_loop(
                0, nv2, 1, unroll=2,
                carry=tuple(jnp.zeros((L,), jnp.int32)
                            for _ in range(7)))(body)
            c = [jnp.sum(a) for a in accs]
            cumb = jnp.int32(0)
            binv = jnp.int32(7)
            kthn = kth - (c[0] + c[1] + c[2] + c[3] + c[4] + c[5] + c[6])
            done = kth <= jnp.int32(0)
            for j in range(7):
                hit = (~done) & (kth <= cumb + c[j])
                binv = jnp.where(hit, jnp.int32(j), binv)
                kthn = jnp.where(hit, kth - cumb, kthn)
                done = done | hit
                cumb = cumb + c[j]
            kth = kthn
            p = p | lax.shift_left(binv, jnp.int32(shift))
    need = kth

    def compact(i, carry):
        sel_base, eq_base = carry
        u = keys[pl.ds(i * L, L)]
        m_lt = u < p
        m_eq = u == p
        eq_i = _ones_where(m_eq)
        excl_eq = plsc.cumsum(eq_i) - eq_i
        m_sel = m_lt | (m_eq & ((eq_base + excl_eq) < need))
        sel_i = _ones_where(m_sel)
        pos = jnp.maximum(sel_base + plsc.cumsum(sel_i) - 1, 0)
        iv = lax.iota(jnp.int32, L) + i * L
        plsc.store_scatter(ckey, [pos], u, mask=m_sel)
        plsc.store_scatter(cidx, [pos], iv, mask=m_sel)
        return (sel_base + plsc.all_reduce_population_count(m_sel),
                eq_base + plsc.all_reduce_population_count(m_eq))

    with jax.named_scope("kmin_compact_sel"):
        plsc.parallel_loop(
            0, NV, 1, unroll=UNROLL,
            carry=(jnp.zeros((L,), jnp.int32),
                   jnp.zeros((L,), jnp.int32)))(compact)

    base_row = wid * T

    def rank_tv(tv, _):
        tkey = ckey[pl.ds(tv * L, L)]

        def rank_le(sv, rank):
            for j in range(L):
                bkey = plsc.load_gather(
                    ckey, [jnp.zeros((L,), jnp.int32) + (sv * L + j)])
                rank = rank + _ones_where(bkey <= tkey)
            return rank

        def rank_lt(sv, rank):
            for j in range(L):
                bkey = plsc.load_gather(
                    ckey, [jnp.zeros((L,), jnp.int32) + (sv * L + j)])
                rank = rank + _ones_where(bkey < tkey)
            return rank

        rank = plsc.parallel_loop(
            0, tv, 1, unroll=2,
            carry=jnp.zeros((L,), jnp.int32))(rank_le)
        rank = plsc.parallel_loop(
            tv + 1, KV, 1, unroll=2, carry=rank)(rank_lt)
        lane = lax.iota(jnp.int32, L)
        for j in range(L):
            bkey = plsc.load_gather(
                ckey, [jnp.zeros((L,), jnp.int32) + (tv * L + j)])
            rank = rank + _ones_where(bkey < tkey)
            rank = rank + _ones_where((bkey == tkey) & (lane > j))
        tval = plsc.bitcast(
            jnp.where(tkey < 0, tkey ^ jnp.int32(0x7FFFFFFF), tkey),
            jnp.float32)
        plsc.store_scatter(sval, [rank], tval)
        plsc.store_scatter(srid, [rank], cidx[pl.ds(tv * L, L)] + base_row)
        return 0

    with jax.named_scope("kmin_rank"):
        lax.fori_loop(0, KV, rank_tv, 0, unroll=False)

    pltpu.sync_copy(sval, vals_out.at[wid])
    for c in range(NCH):
        pltpu.async_copy(
            x_hbm.at[srid.at[pl.ds(c * CH, CH)]], rbuf, gsem).wait()
        pltpu.sync_copy(rbuf, rows_out.at[wid, pl.ds(c * CH, CH)])


def kernel(s, x):
    s2 = s.reshape(B, T)
    xf = x.reshape(B * T, D)
    svals, xrows = _sc_kmin_gather(s2, xf)
    return jnp.concatenate([svals[:, :, None], xrows], axis=-1)


# speedup vs baseline: 1.0075x; 1.0075x over previous
import functools

import jax
import jax.numpy as jnp
from jax import lax
from jax.experimental import pallas as pl
from jax.experimental.pallas import tpu as pltpu
from jax.experimental.pallas import tpu_sc as plsc

B = 32
T = 8192
D = 1024
KSEL = 256
L = 16
NC, NS = 2, 16
NW = NC * NS
NV = T // L
KV = KSEL // L
CH = 64
NCH = KSEL // CH
UNROLL = 8
HI_BITS = 7
INT_MIN = -2147483648

_mesh = plsc.VectorSubcoreMesh(
    core_axis_name="c", subcore_axis_name="s", num_cores=NC, num_subcores=NS
)


def _ones_where(m):
    return jnp.where(m, jnp.int32(1), jnp.int32(0))


@functools.partial(
    pl.kernel,
    out_type=(
        jax.ShapeDtypeStruct((B, KSEL), jnp.float32),
        jax.ShapeDtypeStruct((B, KSEL, D), jnp.float32),
    ),
    mesh=_mesh,
    compiler_params=pltpu.CompilerParams(needs_layout_passes=False),
    scratch_types=[
        pltpu.VMEM((T,), jnp.float32),
        pltpu.VMEM((T,), jnp.int32),
        pltpu.VMEM((T + L,), jnp.int32),
        pltpu.VMEM((KSEL,), jnp.int32),
        pltpu.VMEM((KSEL,), jnp.int32),
        pltpu.VMEM((KSEL,), jnp.float32),
        pltpu.VMEM((KSEL,), jnp.int32),
        pltpu.VMEM((CH, D), jnp.float32),
        pltpu.SemaphoreType.DMA,
    ],
)
def _sc_kmin_gather(s_hbm, x_hbm, vals_out, rows_out,
                    srow, keys, cand, ckey, cidx, sval, srid,
                    rbuf, gsem):
    wid = lax.axis_index("s") * NC + lax.axis_index("c")

    pltpu.sync_copy(s_hbm.at[wid], srow)

    def build_keys(i, acc):
        f = srow[pl.ds(i * L, L)]
        f = jnp.where(f == 0.0, jnp.float32(0.0), f)
        bi = plsc.bitcast(f, jnp.int32)
        u = jnp.where(bi < 0, bi ^ jnp.int32(0x7FFFFFFF), bi)
        keys[pl.ds(i * L, L)] = u
        return acc + _ones_where(u < 0)

    with jax.named_scope("kmin_keys"):
        c_neg = jnp.sum(plsc.parallel_loop(
            0, NV, 1, unroll=UNROLL,
            carry=jnp.zeros((L,), jnp.int32))(build_keys))

    def count_where(cmp_fn):
        def body(i, acc):
            u = keys[pl.ds(i * L, L)]
            return acc + _ones_where(cmp_fn(u))

        acc = plsc.parallel_loop(0, NV, 1, unroll=UNROLL,
                                 carry=jnp.zeros((L,), jnp.int32))(body)
        return jnp.sum(acc)

    kth = jnp.int32(KSEL)
    p = jnp.where(kth <= c_neg, jnp.int32(INT_MIN), jnp.int32(0))
    kth = jnp.where(kth <= c_neg, kth, kth - c_neg)
    with jax.named_scope("kmin_descent_hi"):
        for t in range(30, 30 - HI_BITS, -1):
            pt = lax.shift_right_arithmetic(p, t)
            c0 = count_where(
                lambda u, pt=pt, t=t: lax.shift_right_arithmetic(u, t) == pt)
            keep = kth <= c0
            p = jnp.where(keep, p, p | jnp.int32(1 << t))
            kth = jnp.where(keep, kth, kth - c0)

    tcut = 30 - HI_BITS + 1
    pcut = lax.shift_right_arithmetic(p, tcut)

    def ccand(i, base):
        u = keys[pl.ds(i * L, L)]
        m = lax.shift_right_arithmetic(u, tcut) == pcut
        mi = _ones_where(m)
        pos = jnp.maximum(base + plsc.cumsum(mi) - 1, 0)
        plsc.store_scatter(cand, [pos], u, mask=m)
        return base + plsc.all_reduce_population_count(m)

    with jax.named_scope("kmin_compact_cand"):
        cnt_v = plsc.parallel_loop(
            0, NV, 1, unroll=UNROLL,
            carry=jnp.zeros((L,), jnp.int32))(ccand)
        plsc.store_scatter(cand, [cnt_v + lax.iota(jnp.int32, L)],
                           jnp.zeros((L,), jnp.int32)
                           + (p ^ jnp.int32(INT_MIN)))
    cnt = jnp.max(cnt_v)
    nv2 = lax.div(cnt + jnp.int32(L - 1), jnp.int32(L))
    with jax.named_scope("kmin_descent_lo"):
        for shift in range(tcut - 3, -1, -3):
            base3 = lax.shift_right_arithmetic(p, shift)

            def body(i, acc, base3=base3, shift=shift):
                u = cand[pl.ds(i * L, L)]
                us = lax.shift_right_arithmetic(u, shift)
                return tuple(
                    acc[j] + _ones_where(us == (base3 | j))
                    for j in range(7))

            accs = lax.fori_loop(
                0, nv2, body,
                tuple(jnp.zeros((L,), jnp.int32) for _ in range(7)),
                unroll=False)
            c = [jnp.sum(a) for a in accs]
            cumb = jnp.int32(0)
            binv = jnp.int32(7)
            kthn = kth - (c[0] + c[1] + c[2] + c[3] + c[4] + c[5] + c[6])
            done = kth <= jnp.int32(0)
            for j in range(7):
                hit = (~done) & (kth <= cumb + c[j])
                binv = jnp.where(hit, jnp.int32(j), binv)
                kthn = jnp.where(hit, kth - cumb, kthn)
                done = done | hit
                cumb = cumb + c[j]
            kth = kthn
            p = p | lax.shift_left(binv, jnp.int32(shift))
    need = kth

    def compact(i, carry):
        sel_base, eq_base = carry
        u = keys[pl.ds(i * L, L)]
        m_lt = u < p
        m_eq = u == p
        eq_i = _ones_where(m_eq)
        excl_eq = plsc.cumsum(eq_i) - eq_i
        m_sel = m_lt | (m_eq & ((eq_base + excl_eq) < need))
        sel_i = _ones_where(m_sel)
        pos = jnp.maximum(sel_base + plsc.cumsum(sel_i) - 1, 0)
        iv = lax.iota(jnp.int32, L) + i * L
        plsc.store_scatter(ckey, [pos], u, mask=m_sel)
        plsc.store_scatter(cidx, [pos], iv, mask=m_sel)
        return (sel_base + plsc.all_reduce_population_count(m_sel),
                eq_base + plsc.all_reduce_population_count(m_eq))

    with jax.named_scope("kmin_compact_sel"):
        plsc.parallel_loop(
            0, NV, 1, unroll=UNROLL,
            carry=(jnp.zeros((L,), jnp.int32),
                   jnp.zeros((L,), jnp.int32)))(compact)

    base_row = wid * T

    def rank_tv(tv, _):
        tkey = ckey[pl.ds(tv * L, L)]

        def rank_le(sv, rank):
            for j in range(L):
                bkey = plsc.load_gather(
                    ckey, [jnp.zeros((L,), jnp.int32) + (sv * L + j)])
                rank = rank + _ones_where(bkey <= tkey)
            return rank

        def rank_lt(sv, rank):
            for j in range(L):
                bkey = plsc.load_gather(
                    ckey, [jnp.zeros((L,), jnp.int32) + (sv * L + j)])
                rank = rank + _ones_where(bkey < tkey)
            return rank

        rank = lax.fori_loop(0, tv, rank_le, jnp.zeros((L,), jnp.int32),
                             unroll=False)
        rank = lax.fori_loop(tv + 1, KV, rank_lt, rank, unroll=False)
        lane = lax.iota(jnp.int32, L)
        for j in range(L):
            bkey = plsc.load_gather(
                ckey, [jnp.zeros((L,), jnp.int32) + (tv * L + j)])
            rank = rank + _ones_where(bkey < tkey)
            rank = rank + _ones_where((bkey == tkey) & (lane > j))
        tval = plsc.bitcast(
            jnp.where(tkey < 0, tkey ^ jnp.int32(0x7FFFFFFF), tkey),
            jnp.float32)
        plsc.store_scatter(sval, [rank], tval)
        plsc.store_scatter(srid, [rank], cidx[pl.ds(tv * L, L)] + base_row)
        return 0

    with jax.named_scope("kmin_rank"):
        lax.fori_loop(0, KV, rank_tv, 0, unroll=False)

    pltpu.sync_copy(sval, vals_out.at[wid])
    for c in range(NCH):
        pltpu.async_copy(
            x_hbm.at[srid.at[pl.ds(c * CH, CH)]], rbuf, gsem).wait()
        pltpu.sync_copy(rbuf, rows_out.at[wid, pl.ds(c * CH, CH)])


def kernel(s, x):
    s2 = s.reshape(B, T)
    xf = x.reshape(B * T, D)
    svals, xrows = _sc_kmin_gather(s2, xf)
    return jnp.concatenate([svals[:, :, None], xrows], axis=-1)


# speedup vs baseline: 1.0084x; 1.0009x over previous
import functools

import jax
import jax.numpy as jnp
from jax import lax
from jax.experimental import pallas as pl
from jax.experimental.pallas import tpu as pltpu
from jax.experimental.pallas import tpu_sc as plsc

B = 32
T = 8192
D = 1024
KSEL = 256
L = 16
NC, NS = 2, 16
NW = NC * NS
NV = T // L
KV = KSEL // L
CH = 64
NCH = KSEL // CH
UNROLL = 8
HI_BITS = 7
INT_MIN = -2147483648

_mesh = plsc.VectorSubcoreMesh(
    core_axis_name="c", subcore_axis_name="s", num_cores=NC, num_subcores=NS
)


def _ones_where(m):
    return jnp.where(m, jnp.int32(1), jnp.int32(0))


@functools.partial(
    pl.kernel,
    out_type=(
        jax.ShapeDtypeStruct((B, KSEL), jnp.float32),
        jax.ShapeDtypeStruct((B, KSEL, D), jnp.float32),
    ),
    mesh=_mesh,
    compiler_params=pltpu.CompilerParams(needs_layout_passes=False),
    scratch_types=[
        pltpu.VMEM((T,), jnp.float32),
        pltpu.VMEM((T,), jnp.int32),
        pltpu.VMEM((T + L,), jnp.int32),
        pltpu.VMEM((KSEL,), jnp.int32),
        pltpu.VMEM((KSEL,), jnp.int32),
        pltpu.VMEM((KSEL,), jnp.float32),
        pltpu.VMEM((KSEL,), jnp.int32),
        pltpu.VMEM((CH, D), jnp.float32),
        pltpu.SemaphoreType.DMA,
    ],
)
def _sc_kmin_gather(s_hbm, x_hbm, vals_out, rows_out,
                    srow, keys, cand, ckey, cidx, sval, srid,
                    rbuf, gsem):
    wid = lax.axis_index("s") * NC + lax.axis_index("c")

    pltpu.sync_copy(s_hbm.at[wid], srow)

    def build_keys(i, acc):
        f = srow[pl.ds(i * L, L)]
        f = jnp.where(f == 0.0, jnp.float32(0.0), f)
        bi = plsc.bitcast(f, jnp.int32)
        u = jnp.where(bi < 0, bi ^ jnp.int32(0x7FFFFFFF), bi)
        keys[pl.ds(i * L, L)] = u
        return acc + _ones_where(u < 0)

    with jax.named_scope("kmin_keys"):
        c_neg = jnp.sum(plsc.parallel_loop(
            0, NV, 1, unroll=UNROLL,
            carry=jnp.zeros((L,), jnp.int32))(build_keys))

    def count_where(cmp_fn):
        def body(i, acc):
            u = keys[pl.ds(i * L, L)]
            return acc + _ones_where(cmp_fn(u))

        acc = plsc.parallel_loop(0, NV, 1, unroll=UNROLL,
                                 carry=jnp.zeros((L,), jnp.int32))(body)
        return jnp.sum(acc)

    kth = jnp.int32(KSEL)
    p = jnp.where(kth <= c_neg, jnp.int32(INT_MIN), jnp.int32(0))
    kth = jnp.where(kth <= c_neg, kth, kth - c_neg)
    with jax.named_scope("kmin_descent_hi"):
        for t in range(30, 30 - HI_BITS, -1):
            pt = lax.shift_right_arithmetic(p, t)
            c0 = count_where(
                lambda u, pt=pt, t=t: lax.shift_right_arithmetic(u, t) == pt)
            keep = kth <= c0
            p = jnp.where(keep, p, p | jnp.int32(1 << t))
            kth = jnp.where(keep, kth, kth - c0)

    tcut = 30 - HI_BITS + 1
    pcut = lax.shift_right_arithmetic(p, tcut)

    def ccand(i, base):
        u = keys[pl.ds(i * L, L)]
        m = lax.shift_right_arithmetic(u, tcut) == pcut
        mi = _ones_where(m)
        pos = jnp.maximum(base + plsc.cumsum(mi) - 1, 0)
        plsc.store_scatter(cand, [pos], u, mask=m)
        return base + plsc.all_reduce_population_count(m)

    with jax.named_scope("kmin_compact_cand"):
        cnt_v = plsc.parallel_loop(
            0, NV, 1, unroll=UNROLL,
            carry=jnp.zeros((L,), jnp.int32))(ccand)
        plsc.store_scatter(cand, [cnt_v + lax.iota(jnp.int32, L)],
                           jnp.zeros((L,), jnp.int32)
                           + (p ^ jnp.int32(INT_MIN)))
    cnt = jnp.max(cnt_v)
    nv2 = lax.div(cnt + jnp.int32(L - 1), jnp.int32(L))
    with jax.named_scope("kmin_descent_lo"):
        for shift in range(tcut - 3, -1, -3):
            base3 = lax.shift_right_arithmetic(p, shift)

            def body(i, acc, base3=base3, shift=shift):
                u = cand[pl.ds(i * L, L)]
                us = lax.shift_right_arithmetic(u, shift)
                return tuple(
                    acc[j] + _ones_where(us == (base3 | j))
                    for j in range(7))

            accs = lax.fori_loop(
                0, nv2, body,
                tuple(jnp.zeros((L,), jnp.int32) for _ in range(7)),
                unroll=False)
            c = [jnp.sum(a) for a in accs]
            cumb = jnp.int32(0)
            binv = jnp.int32(7)
            kthn = kth - (c[0] + c[1] + c[2] + c[3] + c[4] + c[5] + c[6])
            done = kth <= jnp.int32(0)
            for j in range(7):
                hit = (~done) & (kth <= cumb + c[j])
                binv = jnp.where(hit, jnp.int32(j), binv)
                kthn = jnp.where(hit, kth - cumb, kthn)
                done = done | hit
                cumb = cumb + c[j]
            kth = kthn
            p = p | lax.shift_left(binv, jnp.int32(shift))
    need = kth

    def compact(i, carry):
        sel_base, eq_base = carry
        u = keys[pl.ds(i * L, L)]
        m_lt = u < p
        m_eq = u == p
        eq_i = _ones_where(m_eq)
        excl_eq = plsc.cumsum(eq_i) - eq_i
        m_sel = m_lt | (m_eq & ((eq_base + excl_eq) < need))
        sel_i = _ones_where(m_sel)
        pos = jnp.maximum(sel_base + plsc.cumsum(sel_i) - 1, 0)
        iv = lax.iota(jnp.int32, L) + i * L
        plsc.store_scatter(ckey, [pos], u, mask=m_sel)
        plsc.store_scatter(cidx, [pos], iv, mask=m_sel)
        return (sel_base + plsc.all_reduce_population_count(m_sel),
                eq_base + plsc.all_reduce_population_count(m_eq))

    with jax.named_scope("kmin_compact_sel"):
        plsc.parallel_loop(
            0, NV, 1, unroll=UNROLL,
            carry=(jnp.zeros((L,), jnp.int32),
                   jnp.zeros((L,), jnp.int32)))(compact)

    base_row = wid * T

    def rank_tv(tv, _):
        tkey = ckey[pl.ds(tv * L, L)]

        def rank_le(sv, rank):
            for j in range(L):
                bkey = plsc.load_gather(
                    ckey, [jnp.zeros((L,), jnp.int32) + (sv * L + j)])
                rank = rank + _ones_where(bkey <= tkey)
            return rank

        def rank_lt(sv, rank):
            for j in range(L):
                bkey = plsc.load_gather(
                    ckey, [jnp.zeros((L,), jnp.int32) + (sv * L + j)])
                rank = rank + _ones_where(bkey < tkey)
            return rank

        rank = plsc.parallel_loop(
            0, tv, 1, unroll=1,
            carry=jnp.zeros((L,), jnp.int32))(rank_le)
        rank = plsc.parallel_loop(
            tv + 1, KV, 1, unroll=1, carry=rank)(rank_lt)
        lane = lax.iota(jnp.int32, L)
        for j in range(L):
            bkey = plsc.load_gather(
                ckey, [jnp.zeros((L,), jnp.int32) + (tv * L + j)])
            rank = rank + _ones_where(bkey < tkey)
            rank = rank + _ones_where((bkey == tkey) & (lane > j))
        tval = plsc.bitcast(
            jnp.where(tkey < 0, tkey ^ jnp.int32(0x7FFFFFFF), tkey),
            jnp.float32)
        plsc.store_scatter(sval, [rank], tval)
        plsc.store_scatter(srid, [rank], cidx[pl.ds(tv * L, L)] + base_row)
        return 0

    with jax.named_scope("kmin_rank"):
        lax.fori_loop(0, KV, rank_tv, 0, unroll=False)

    pltpu.sync_copy(sval, vals_out.at[wid])
    for c in range(NCH):
        pltpu.async_copy(
            x_hbm.at[srid.at[pl.ds(c * CH, CH)]], rbuf, gsem).wait()
        pltpu.sync_copy(rbuf, rows_out.at[wid, pl.ds(c * CH, CH)])


def kernel(s, x):
    s2 = s.reshape(B, T)
    xf = x.reshape(B * T, D)
    svals, xrows = _sc_kmin_gather(s2, xf)
    return jnp.concatenate([svals[:, :, None], xrows], axis=-1)
